# Initial kernel scaffold; baseline (speedup 1.0000x reference)
import jax
import jax.numpy as jnp
from jax.experimental import pallas as pl
from jax.experimental.pallas import tpu as pltpu

_B = 16


def _patch_softmax_kernel(x_ref, o_ref):
    v = x_ref[...]
    b, h, w = v.shape
    vw = v.reshape(b, h // 2, 2, w)

    lane = jax.lax.broadcasted_iota(jnp.int32, (b, h // 2, w), 2)
    even = (lane & 1) == 0

    def pair_lanes(t, op):
        swapped = jnp.where(
            even, pltpu.roll(t, -1, axis=2), pltpu.roll(t, 1, axis=2)
        )
        return op(t, swapped)

    m2 = jnp.max(vw, axis=2)
    m = pair_lanes(m2, jnp.maximum)
    e = jnp.exp(vw - m[:, :, None, :])
    s2 = jnp.sum(e, axis=2)
    s = pair_lanes(s2, jnp.add)
    r = 1.0 / s
    o_ref[...] = (e * r[:, :, None, :]).reshape(b, h, w)


def kernel(x):
    b, c, h, w = x.shape
    n = b * c
    xf = x.reshape(n, h, w)
    out = pl.pallas_call(
        _patch_softmax_kernel,
        grid=(n // _B,),
        in_specs=[pl.BlockSpec((_B, h, w), lambda i: (i, 0, 0))],
        out_specs=pl.BlockSpec((_B, h, w), lambda i: (i, 0, 0)),
        out_shape=jax.ShapeDtypeStruct((n, h, w), x.dtype),
        compiler_params=pltpu.CompilerParams(
            dimension_semantics=("parallel",),
        ),
    )(xf)
    return out.reshape(b, c, h, w)


# speedup vs baseline: 1.7255x; 1.7255x over previous
import jax
import jax.numpy as jnp
from jax.experimental import pallas as pl
from jax.experimental.pallas import tpu as pltpu

_B = 16


def _patch_softmax_kernel(x_ref, o_ref):
    v = x_ref[...]
    b, h, w = v.shape
    vw = v.reshape(b, h // 2, 2, w)

    lane = jax.lax.broadcasted_iota(jnp.int32, (b, h // 2, w), 2)
    even = (lane & 1) == 0

    def pair_lanes(t, op):
        swapped = jnp.where(
            even, pltpu.roll(t, t.shape[2] - 1, axis=2), pltpu.roll(t, 1, axis=2)
        )
        return op(t, swapped)

    m2 = jnp.max(vw, axis=2)
    m = pair_lanes(m2, jnp.maximum)
    e = jnp.exp(vw - m[:, :, None, :])
    s2 = jnp.sum(e, axis=2)
    s = pair_lanes(s2, jnp.add)
    r = 1.0 / s
    o_ref[...] = (e * r[:, :, None, :]).reshape(b, h, w)


def kernel(x):
    b, c, h, w = x.shape
    n = b * c
    xf = x.reshape(n, h, w)
    out = pl.pallas_call(
        _patch_softmax_kernel,
        grid=(n // _B,),
        in_specs=[pl.BlockSpec((_B, h, w), lambda i: (i, 0, 0))],
        out_specs=pl.BlockSpec((_B, h, w), lambda i: (i, 0, 0)),
        out_shape=jax.ShapeDtypeStruct((n, h, w), x.dtype),
        compiler_params=pltpu.CompilerParams(
            dimension_semantics=("parallel",),
        ),
    )(xf)
    return out.reshape(b, c, h, w)


# speedup vs baseline: 5.2894x; 3.0655x over previous
import jax
import jax.numpy as jnp
from jax.experimental import pallas as pl
from jax.experimental.pallas import tpu as pltpu

_B = 16


def _patch_softmax_kernel(x_ref, o_ref):
    v = x_ref[...]
    b, h, w = v.shape

    lane = jax.lax.broadcasted_iota(jnp.int32, (b, h, w), 2)
    lane_even = (lane & 1) == 0
    sub = jax.lax.broadcasted_iota(jnp.int32, (b, h, w), 1)
    sub_even = (sub & 1) == 0

    def swap_lanes(t):
        return jnp.where(
            lane_even, pltpu.roll(t, w - 1, axis=2), pltpu.roll(t, 1, axis=2)
        )

    def swap_sublanes(t):
        return jnp.where(
            sub_even, pltpu.roll(t, h - 1, axis=1), pltpu.roll(t, 1, axis=1)
        )

    sv = swap_lanes(v)
    mx = jnp.maximum(v, sv)
    m = jnp.maximum(mx, swap_sublanes(mx))
    e = jnp.exp(v - m)
    es = e + jnp.exp(sv - m)
    s = es + swap_sublanes(es)
    o_ref[...] = e * (1.0 / s)


def kernel(x):
    b, c, h, w = x.shape
    n = b * c
    xf = x.reshape(n, h, w)
    out = pl.pallas_call(
        _patch_softmax_kernel,
        grid=(n // _B,),
        in_specs=[pl.BlockSpec((_B, h, w), lambda i: (i, 0, 0))],
        out_specs=pl.BlockSpec((_B, h, w), lambda i: (i, 0, 0)),
        out_shape=jax.ShapeDtypeStruct((n, h, w), x.dtype),
        compiler_params=pltpu.CompilerParams(
            dimension_semantics=("parallel",),
        ),
    )(xf)
    return out.reshape(b, c, h, w)
